# Initial kernel scaffold; baseline (speedup 1.0000x reference)
import jax
import jax.numpy as jnp
from jax.experimental import pallas as pl

_D = 256
_S = 512
_MAXLEN = 1023
_ROWS_PER_BLOCK = 8


def _rpe_kernel(flip_ref, out_ref):
    i = pl.program_id(0)
    r0 = i * _ROWS_PER_BLOCK
    for r in range(_ROWS_PER_BLOCK):
        start = (_S - 1) - (r0 + r)
        out_ref[r] = flip_ref[pl.ds(start, _S), :]


def kernel(x, rel_embeddings):
    flipped = rel_embeddings[::-1]
    rel_pos = pl.pallas_call(
        _rpe_kernel,
        grid=(_S // _ROWS_PER_BLOCK,),
        in_specs=[pl.BlockSpec((_MAXLEN, _D), lambda i: (0, 0))],
        out_specs=pl.BlockSpec((_ROWS_PER_BLOCK, _S, _D), lambda i: (i, 0, 0)),
        out_shape=jax.ShapeDtypeStruct((_S, _S, _D), jnp.float32),
    )(flipped)
    return (x, rel_pos)


# speedup vs baseline: 11.2518x; 11.2518x over previous
import jax
import jax.numpy as jnp
from jax.experimental import pallas as pl

_D = 256
_S = 512
_RPB = 8


def _rpe_kernel(a_ref, out_ref):
    i = pl.program_id(0)
    q8 = pl.multiple_of(8 * (_S // _RPB - 1 - i), 8)
    for r in range(_RPB):
        out_ref[r] = a_ref[7 - r, pl.ds(q8, _S), :]


def kernel(x, rel_embeddings):
    flipped = rel_embeddings[::-1]
    flippad = jnp.pad(flipped, ((0, 9), (0, 0)))
    shifted = jnp.stack([flippad[k:k + 2 * _S] for k in range(8)])
    rel_pos = pl.pallas_call(
        _rpe_kernel,
        grid=(_S // _RPB,),
        in_specs=[pl.BlockSpec((8, 2 * _S, _D), lambda i: (0, 0, 0))],
        out_specs=pl.BlockSpec((_RPB, _S, _D), lambda i: (i, 0, 0)),
        out_shape=jax.ShapeDtypeStruct((_S, _S, _D), jnp.float32),
    )(shifted)
    return (x, rel_pos)
